# baseline (speedup 1.0000x reference)
import functools

import jax
import jax.numpy as jnp
from jax import lax
from jax.experimental import pallas as pl
from jax.experimental.pallas import tpu as pltpu
from jax.experimental.pallas import tpu_sc as plsc

_BATCH = 4096
_SEQ = 200
_HIDDEN = 64
_VOCAB = 1000000
_TOTAL = _BATCH * _SEQ
_NW = 32
_CHUNK = 128
_NCHUNK = _TOTAL // (_NW * _CHUNK)
_K = 2
_NSETS = _NCHUNK // _K
_PAIRS = _NSETS // 2
_NBT = _BATCH // _CHUNK


def _make_lookup():
    mesh = plsc.VectorSubcoreMesh(core_axis_name="c", subcore_axis_name="s")

    @functools.partial(
        pl.kernel,
        mesh=mesh,
        out_type=jax.ShapeDtypeStruct((_SEQ, 8, _NBT, 8, _CHUNK),
                                      jnp.float32),
        scratch_types=[
            pltpu.VMEM((_NCHUNK, _CHUNK), jnp.int32),
            pltpu.VMEM((2, _K, _CHUNK, _CHUNK), jnp.float32),
            pltpu.VMEM((2, _K, 64, _CHUNK), jnp.float32),
            pltpu.SemaphoreType.DMA,
            pltpu.SemaphoreType.DMA,
            pltpu.SemaphoreType.DMA,
            pltpu.SemaphoreType.DMA,
        ],
        compiler_params=pltpu.CompilerParams(
            use_tc_tiling_on_sc=False, needs_layout_passes=False),
    )
    def lookup(table_hbm, idx_hbm, out_hbm, idx_v, rows_v, trans_v,
               g0s, g1s, w0s, w1s):
        wid = lax.axis_index("s") * 2 + lax.axis_index("c")
        chunk0 = wid * _NCHUNK
        gsems = (g0s, g1s)
        wsems = (w0s, w1s)

        pltpu.sync_copy(idx_hbm.at[pl.ds(chunk0, _NCHUNK)], idx_v)

        def fire_gathers(s, p):
            for b in range(_K):
                pltpu.async_copy(
                    table_hbm.at[idx_v.at[s * _K + b]],
                    rows_v.at[p].at[b],
                    gsems[p],
                )

        def drain_gathers(sem):
            for b in range(_K):
                pltpu.make_async_copy(
                    table_hbm.at[pl.ds(0, _CHUNK)],
                    rows_v.at[0].at[b],
                    sem,
                ).wait()

        def drain_writebacks(sem):
            for b in range(_K):
                for ti in range(8):
                    pltpu.make_async_copy(
                        trans_v.at[0].at[b].at[pl.ds(ti * 8, 8)],
                        out_hbm.at[0].at[0].at[0],
                        sem,
                    ).wait()

        def select_transpose_writeback(s, p):
            for b in range(_K):
                rows = rows_v.at[p].at[b]
                trans = trans_v.at[p].at[b]
                c = chunk0 + s * _K + b
                sq = c // _NBT
                tj = lax.rem(c, _NBT)
                for j in range(8):
                    bsel = lax.iota(jnp.int32, 16) + (j * 16)
                    zeros = lax.iota(jnp.int32, 16) * 0

                    def body(f, carry, j=j, bsel=bsel, zeros=zeros,
                             rows=rows, trans=trans):
                        vals = plsc.load_gather(rows, [bsel, zeros + f])
                        trans.at[f][pl.ds(j * 16, 16)] = vals
                        return carry

                    lax.fori_loop(0, 64, body, 0)
                for ti in range(8):
                    pltpu.async_copy(
                        trans.at[pl.ds(ti * 8, 8)],
                        out_hbm.at[sq].at[ti].at[tj],
                        wsems[p],
                    )

        fire_gathers(0, 0)

        def pair(t, carry):
            for p in range(2):
                s = 2 * t + p
                if p == 0:
                    @pl.when(t > 0)
                    def _():
                        drain_writebacks(wsems[1])
                else:
                    drain_writebacks(wsems[0])
                if p == 0:
                    fire_gathers(s + 1, 1)
                else:
                    @pl.when(t < _PAIRS - 1)
                    def _():
                        fire_gathers(s + 1, 0)
                drain_gathers(gsems[p])
                select_transpose_writeback(s, p)
            return carry

        lax.fori_loop(0, _PAIRS, pair, 0)

        drain_writebacks(wsems[1])

    return lookup


_lookup = _make_lookup()


@jax.jit
def kernel(item_seqs, emb):
    table3 = jnp.pad(emb, ((0, 0), (0, 2 * _HIDDEN - _HIDDEN)))
    flat_idx = jnp.transpose(item_seqs).reshape(_TOTAL // _CHUNK, _CHUNK)
    out5 = _lookup(table3, flat_idx)
    return jnp.transpose(out5, (2, 4, 0, 1, 3)).reshape(
        _BATCH, _SEQ, _HIDDEN)


# speedup vs baseline: 1.0001x; 1.0001x over previous
import functools

import jax
import jax.numpy as jnp
from jax import lax
from jax.experimental import pallas as pl
from jax.experimental.pallas import tpu as pltpu
from jax.experimental.pallas import tpu_sc as plsc

_BATCH = 4096
_SEQ = 200
_HIDDEN = 64
_VOCAB = 1000000
_TOTAL = _BATCH * _SEQ
_NW = 32
_CHUNK = 128
_NCHUNK = _TOTAL // (_NW * _CHUNK)
_K = 2
_NSETS = _NCHUNK // _K
_PAIRS = _NSETS // 2
_NBT = _BATCH // _CHUNK


def _make_lookup():
    mesh = plsc.VectorSubcoreMesh(core_axis_name="c", subcore_axis_name="s")

    @functools.partial(
        pl.kernel,
        mesh=mesh,
        out_type=jax.ShapeDtypeStruct((_SEQ, 8, _NBT, 8, _CHUNK),
                                      jnp.float32),
        scratch_types=[
            pltpu.VMEM((_NCHUNK, _CHUNK), jnp.int32),
            pltpu.VMEM((2, _K, _CHUNK, _CHUNK), jnp.float32),
            pltpu.VMEM((2, _K, 64, _CHUNK), jnp.float32),
            pltpu.SemaphoreType.DMA,
            pltpu.SemaphoreType.DMA,
            pltpu.SemaphoreType.DMA,
            pltpu.SemaphoreType.DMA,
        ],
        compiler_params=pltpu.CompilerParams(
            use_tc_tiling_on_sc=False, needs_layout_passes=False),
    )
    def lookup(table_hbm, idx_hbm, out_hbm, idx_v, rows_v, trans_v,
               g0s, g1s, w0s, w1s):
        wid = lax.axis_index("s") * 2 + lax.axis_index("c")
        chunk0 = wid * _NCHUNK
        gsems = (g0s, g1s)
        wsems = (w0s, w1s)

        pltpu.sync_copy(idx_hbm.at[pl.ds(chunk0, _NCHUNK)], idx_v)

        def fire_gathers(s, p):
            for b in range(_K):
                pltpu.async_copy(
                    table_hbm.at[idx_v.at[s * _K + b]],
                    rows_v.at[p].at[b],
                    gsems[p],
                )

        def drain_gathers(sem):
            for b in range(_K):
                pltpu.make_async_copy(
                    table_hbm.at[pl.ds(0, _CHUNK)],
                    rows_v.at[0].at[b],
                    sem,
                ).wait()

        def drain_writebacks(sem):
            for b in range(_K):
                for ti in range(8):
                    pltpu.make_async_copy(
                        trans_v.at[0].at[b].at[pl.ds(ti * 8, 8)],
                        out_hbm.at[0].at[0].at[0],
                        sem,
                    ).wait()

        def select_transpose_writeback(s, p):
            for b in range(_K):
                rows = rows_v.at[p].at[b]
                trans = trans_v.at[p].at[b]
                c = chunk0 + s * _K + b
                sq = c // _NBT
                tj = lax.rem(c, _NBT)
                lane = lax.iota(jnp.int32, 16)

                def body(f, carry, rows=rows, trans=trans, lane=lane):
                    fvec = lane * 0 + f
                    trow = trans.at[f]
                    for j in range(8):
                        vals = plsc.load_gather(
                            rows, [lane + (j * 16), fvec])
                        trow[pl.ds(j * 16, 16)] = vals
                    return carry

                lax.fori_loop(0, 64, body, 0)
                for ti in range(8):
                    pltpu.async_copy(
                        trans.at[pl.ds(ti * 8, 8)],
                        out_hbm.at[sq].at[ti].at[tj],
                        wsems[p],
                    )

        fire_gathers(0, 0)

        def pair(t, carry):
            for p in range(2):
                s = 2 * t + p
                if p == 0:
                    @pl.when(t > 0)
                    def _():
                        drain_writebacks(wsems[1])
                else:
                    drain_writebacks(wsems[0])
                if p == 0:
                    fire_gathers(s + 1, 1)
                else:
                    @pl.when(t < _PAIRS - 1)
                    def _():
                        fire_gathers(s + 1, 0)
                drain_gathers(gsems[p])
                select_transpose_writeback(s, p)
            return carry

        lax.fori_loop(0, _PAIRS, pair, 0)

        drain_writebacks(wsems[1])

    return lookup


_lookup = _make_lookup()


@jax.jit
def kernel(item_seqs, emb):
    table3 = jnp.pad(emb, ((0, 0), (0, 2 * _HIDDEN - _HIDDEN)))
    flat_idx = jnp.transpose(item_seqs).reshape(_TOTAL // _CHUNK, _CHUNK)
    out5 = _lookup(table3, flat_idx)
    return jnp.transpose(out5, (2, 4, 0, 1, 3)).reshape(
        _BATCH, _SEQ, _HIDDEN)


# speedup vs baseline: 1.7774x; 1.7773x over previous
import functools

import jax
import jax.numpy as jnp
from jax import lax
from jax.experimental import pallas as pl
from jax.experimental.pallas import tpu as pltpu
from jax.experimental.pallas import tpu_sc as plsc

_BATCH = 4096
_SEQ = 200
_HIDDEN = 64
_VOCAB = 1000000
_PADW = 128
_TOTAL = _BATCH * _SEQ
_NW = 32
_CHUNK = 128
_NCHUNK = _TOTAL // (_NW * _CHUNK)
_K = 2
_NSETS = _NCHUNK // _K
_PAIRS = _NSETS // 2


def _make_lookup():
    mesh = plsc.VectorSubcoreMesh(core_axis_name="c", subcore_axis_name="s")

    @functools.partial(
        pl.kernel,
        mesh=mesh,
        out_type=jax.ShapeDtypeStruct((_TOTAL, _PADW), jnp.float32),
        scratch_types=[
            pltpu.VMEM((_NCHUNK, _CHUNK), jnp.int32),
            pltpu.VMEM((2, _K, _CHUNK, _PADW), jnp.float32),
            pltpu.SemaphoreType.DMA,
            pltpu.SemaphoreType.DMA,
            pltpu.SemaphoreType.DMA,
            pltpu.SemaphoreType.DMA,
        ],
        compiler_params=pltpu.CompilerParams(
            use_tc_tiling_on_sc=False, needs_layout_passes=False),
    )
    def lookup(table_hbm, idx_hbm, out_hbm, idx_v, rows_v, g0s, g1s, w0s,
               w1s):
        wid = lax.axis_index("s") * 2 + lax.axis_index("c")
        chunk0 = wid * _NCHUNK
        gsems = (g0s, g1s)
        wsems = (w0s, w1s)

        pltpu.sync_copy(idx_hbm.at[pl.ds(chunk0, _NCHUNK)], idx_v)

        def fire_gathers(s, p):
            for b in range(_K):
                pltpu.async_copy(
                    table_hbm.at[idx_v.at[s * _K + b]],
                    rows_v.at[p].at[b],
                    gsems[p],
                )

        def drain(sem, dst_vmem):
            for b in range(_K):
                if dst_vmem:
                    pltpu.make_async_copy(
                        table_hbm.at[pl.ds(0, _CHUNK)],
                        rows_v.at[0].at[b],
                        sem,
                    ).wait()
                else:
                    pltpu.make_async_copy(
                        rows_v.at[0].at[b],
                        out_hbm.at[pl.ds(0, _CHUNK)],
                        sem,
                    ).wait()

        def fire_writebacks(s, p):
            for b in range(_K):
                ga = (chunk0 + s * _K + b) * _CHUNK
                pltpu.async_copy(
                    rows_v.at[p].at[b],
                    out_hbm.at[pl.ds(ga, _CHUNK)],
                    wsems[p],
                )

        fire_gathers(0, 0)

        def pair(t, carry):
            for p in range(2):
                s = 2 * t + p
                if p == 0:
                    @pl.when(t > 0)
                    def _():
                        drain(wsems[1], False)
                else:
                    drain(wsems[0], False)
                if p == 0:
                    fire_gathers(s + 1, 1)
                else:
                    @pl.when(t < _PAIRS - 1)
                    def _():
                        fire_gathers(s + 1, 0)
                drain(gsems[p], True)
                fire_writebacks(s, p)
            return carry

        lax.fori_loop(0, _PAIRS, pair, 0)

        drain(wsems[1], False)

    return lookup


_lookup = _make_lookup()


def kernel(item_seqs, emb):
    table3 = jnp.pad(emb, ((0, 0), (0, _PADW - _HIDDEN)))
    flat_idx = item_seqs.reshape(_TOTAL // _CHUNK, _CHUNK)
    out = _lookup(table3, flat_idx)
    return out.reshape(_BATCH, _SEQ, _PADW)[:, :, :_HIDDEN]
